# Initial kernel scaffold; baseline (speedup 1.0000x reference)
import jax
import jax.numpy as jnp
from jax.experimental import pallas as pl


def kernel(h0, h1, edge_index0, edge_index1, params):
    raise NotImplementedError("write your pallas kernel here")


# speedup vs baseline: 1.8713x; 1.8713x over previous
import functools

import jax
import jax.numpy as jnp
from jax.experimental import pallas as pl

N0 = 10000
E0 = 160000
E1 = 1280000
D = 128


def _dense_z_body(h_ref, g_ref, wc_ref, wf_ref, ns_ref, z_ref):
    acc = jnp.dot(h_ref[...], wc_ref[...], preferred_element_type=jnp.float32)
    acc += jnp.dot(g_ref[...], wf_ref[...], preferred_element_type=jnp.float32)
    z_ref[...] = acc * ns_ref[...]


def _dense_z(h, g, wc, wf, ns, blk):
    n = h.shape[0]
    grid = n // blk
    return pl.pallas_call(
        _dense_z_body,
        grid=(grid,),
        in_specs=[
            pl.BlockSpec((blk, D), lambda i: (i, 0)),
            pl.BlockSpec((blk, D), lambda i: (i, 0)),
            pl.BlockSpec((D, D), lambda i: (0, 0)),
            pl.BlockSpec((D, D), lambda i: (0, 0)),
            pl.BlockSpec((blk, 1), lambda i: (i, 0)),
        ],
        out_specs=pl.BlockSpec((blk, D), lambda i: (i, 0)),
        out_shape=jax.ShapeDtypeStruct((n, D), jnp.float32),
    )(h, g, wc, wf, ns)


def _epilogue_body(agg_ref, z_ref, nd_ref, b_ref, gamma_ref, beta_ref, o_ref):
    pre = (agg_ref[...] + z_ref[...]) * nd_ref[...] + b_ref[...]
    mu = jnp.mean(pre, axis=-1, keepdims=True)
    var = jnp.mean((pre - mu) ** 2, axis=-1, keepdims=True)
    ln = (pre - mu) * jax.lax.rsqrt(var + 1e-5) * gamma_ref[...] + beta_ref[...]
    o_ref[...] = jnp.maximum(ln, 0.0)


def _epilogue(agg, z, nd, b, gamma, beta, blk):
    n = agg.shape[0]
    grid = n // blk
    return pl.pallas_call(
        _epilogue_body,
        grid=(grid,),
        in_specs=[
            pl.BlockSpec((blk, D), lambda i: (i, 0)),
            pl.BlockSpec((blk, D), lambda i: (i, 0)),
            pl.BlockSpec((blk, 1), lambda i: (i, 0)),
            pl.BlockSpec((1, D), lambda i: (0, 0)),
            pl.BlockSpec((1, D), lambda i: (0, 0)),
            pl.BlockSpec((1, D), lambda i: (0, 0)),
        ],
        out_specs=pl.BlockSpec((blk, D), lambda i: (i, 0)),
        out_shape=jax.ShapeDtypeStruct((n, D), jnp.float32),
    )(agg, z, nd, b, gamma, beta)


def _level(h, g, src, dst, n, p, blk):
    e = src.shape[0]
    ones = jnp.ones(e, jnp.float32)
    deg_out = jax.ops.segment_sum(ones, src, num_segments=n) + 1.0
    deg_in = jax.ops.segment_sum(ones, dst, num_segments=n) + 1.0
    ns = deg_out ** -0.5
    nd = deg_in ** -0.5
    wc = p['Wc'] * p['conv_w'][None, :]
    wf = p['Wf'] * p['fuse_w'][None, :]
    b = p['bc'] * p['conv_w'] + p['bf'] * p['fuse_w']
    z = _dense_z(h, g, wc, wf, ns[:, None], blk)
    agg = jax.ops.segment_sum(jnp.take(z, src, axis=0), dst, num_segments=n)
    return _epilogue(agg, z, nd[:, None], b[None, :], p['gamma'][None, :],
                     p['beta'][None, :], blk)


def kernel(h0, h1, edge_index0, edge_index1, params):
    src0, dst0 = edge_index0[0], edge_index0[1]
    src1, dst1 = edge_index1[0], edge_index1[1]
    inc_msg = jax.ops.segment_sum(h1, dst0, num_segments=N0)
    r0 = _level(h0, inc_msg, src0, dst0, N0, params['td'], 400)
    gath = jnp.take(h0, dst0, axis=0)
    r1 = _level(h1, gath, src1, dst1, E0, params['bu'], 2000)
    return (r0, r1)


# speedup vs baseline: 2.8636x; 1.5303x over previous
import functools

import jax
import jax.numpy as jnp
from jax import lax
from jax.experimental import pallas as pl
from jax.experimental.pallas import tpu as pltpu
from jax.experimental.pallas import tpu_sc as plsc

N0 = 10000
E0 = 160000
E1 = 1280000
D = 128

_NC = 2
_NS = 16
_NW = _NC * _NS

_mesh = plsc.VectorSubcoreMesh(core_axis_name="c", subcore_axis_name="s")


def _sc_hist2(idx_a, idx_b, nbins_pad):
    nrow = idx_a.shape[0]
    units = nrow // 8
    q, r = units // _NS, units % _NS
    stripe = nbins_pad // _NS

    @functools.partial(
        pl.kernel,
        out_type=jax.ShapeDtypeStruct((2, nbins_pad), jnp.float32),
        mesh=_mesh,
        scratch_types=[
            pltpu.VMEM((8, 128), jnp.int32),
            pltpu.VMEM((128,), jnp.float32),
            pltpu.VMEM((stripe,), jnp.float32),
            pltpu.VMEM_SHARED((nbins_pad,), jnp.float32),
        ],
    )
    def k(a_hbm, b_hbm, out_hbm, idx_v, ones_v, zero_v, acc_sh):
        cid = lax.axis_index("c")
        tid = lax.axis_index("s")

        def fill(i, _):
            zero_v[pl.ds(i * 16, 16)] = jnp.zeros((16,), jnp.float32)
            return 0

        lax.fori_loop(0, stripe // 16, fill, 0)
        for i in range(8):
            ones_v[pl.ds(i * 16, 16)] = jnp.ones((16,), jnp.float32)
        pltpu.sync_copy(zero_v, acc_sh.at[pl.ds(tid * stripe, stripe)])
        plsc.subcore_barrier()

        nunit = q + jnp.where(tid < r, 1, 0)
        base = tid * q + jnp.minimum(tid, r)

        def unit_body(u, _):
            @pl.when(cid == 0)
            def _():
                pltpu.sync_copy(a_hbm.at[pl.ds((base + u) * 8, 8)], idx_v)

            @pl.when(cid == 1)
            def _():
                pltpu.sync_copy(b_hbm.at[pl.ds((base + u) * 8, 8)], idx_v)

            for j in range(8):
                pltpu.sync_copy(ones_v, acc_sh.at[idx_v.at[j]], add=True)
            return 0

        lax.fori_loop(0, nunit, unit_body, 0)
        plsc.subcore_barrier()
        pltpu.sync_copy(acc_sh.at[pl.ds(tid * stripe, stripe)],
                        out_hbm.at[cid].at[pl.ds(tid * stripe, stripe)])

    return k(idx_a, idx_b)


def _sc_gather_inc(h0a, h1a, idx2d):
    nblk_real = E0 // 128
    units_w = idx2d.shape[0] // 8 // _NW

    @functools.partial(
        pl.kernel,
        out_type=(jax.ShapeDtypeStruct((E0, D), jnp.float32),
                  jax.ShapeDtypeStruct((2, N0, D), jnp.float32)),
        mesh=_mesh,
        scratch_types=[
            pltpu.VMEM((8, 128), jnp.int32),
            pltpu.VMEM((128, D), jnp.float32),
            pltpu.VMEM((128, D), jnp.float32),
            pltpu.VMEM_SHARED((N0, D), jnp.float32),
            pltpu.SemaphoreType.DMA,
        ],
    )
    def k(h0_hbm, h1_hbm, idx_hbm, zeros_hbm, gath_hbm, inc_hbm, idx_v,
          grow_v, hrow_v, acc_sh, sem):
        cid = lax.axis_index("c")
        tid = lax.axis_index("s")
        w = tid * _NC + cid

        @pl.when(tid < 10)
        def _():
            pltpu.sync_copy(zeros_hbm.at[pl.ds(tid * 1000, 1000)],
                            acc_sh.at[pl.ds(tid * 1000, 1000)])

        plsc.subcore_barrier()

        def unit_body(u, _):
            ub = w * units_w + u
            pltpu.sync_copy(idx_hbm.at[pl.ds(ub * 8, 8)], idx_v)
            for j in range(8):
                gblk = ub * 8 + j

                @pl.when(gblk < nblk_real)
                def _():
                    pltpu.async_copy(h0_hbm.at[idx_v.at[j]], grow_v, sem).wait()
                    pltpu.sync_copy(grow_v,
                                    gath_hbm.at[pl.ds(gblk * 128, 128)])
                    pltpu.sync_copy(h1_hbm.at[pl.ds(gblk * 128, 128)], hrow_v)
                    pltpu.sync_copy(hrow_v, acc_sh.at[idx_v.at[j]], add=True)
            return 0

        lax.fori_loop(0, units_w, unit_body, 0)
        plsc.subcore_barrier()

        @pl.when(tid < 10)
        def _():
            pltpu.sync_copy(acc_sh.at[pl.ds(tid * 1000, 1000)],
                            inc_hbm.at[cid].at[pl.ds(tid * 1000, 1000)])

    zeros = jnp.zeros((N0, D), jnp.float32)
    return k(h0a, h1a, idx2d, zeros)


def _sc_gather_scatter0(table, src2d, dst2d):
    nblk_real = E0 // 128
    units_w = src2d.shape[0] // 8 // _NW

    @functools.partial(
        pl.kernel,
        out_type=jax.ShapeDtypeStruct((2, N0, D), jnp.float32),
        mesh=_mesh,
        scratch_types=[
            pltpu.VMEM((8, 128), jnp.int32),
            pltpu.VMEM((8, 128), jnp.int32),
            pltpu.VMEM((128, D), jnp.float32),
            pltpu.VMEM_SHARED((N0, D), jnp.float32),
            pltpu.SemaphoreType.DMA,
        ],
    )
    def k(tab_hbm, src_hbm, dst_hbm, zeros_hbm, agg_hbm, src_v, dst_v,
          rows_v, acc_sh, sem):
        cid = lax.axis_index("c")
        tid = lax.axis_index("s")
        w = tid * _NC + cid

        @pl.when(tid < 10)
        def _():
            pltpu.sync_copy(zeros_hbm.at[pl.ds(tid * 1000, 1000)],
                            acc_sh.at[pl.ds(tid * 1000, 1000)])

        plsc.subcore_barrier()

        def unit_body(u, _):
            ub = w * units_w + u
            pltpu.sync_copy(src_hbm.at[pl.ds(ub * 8, 8)], src_v)
            pltpu.sync_copy(dst_hbm.at[pl.ds(ub * 8, 8)], dst_v)
            for j in range(8):
                gblk = ub * 8 + j

                @pl.when(gblk < nblk_real)
                def _():
                    pltpu.async_copy(tab_hbm.at[src_v.at[j]], rows_v, sem).wait()
                    pltpu.sync_copy(rows_v, acc_sh.at[dst_v.at[j]], add=True)
            return 0

        lax.fori_loop(0, units_w, unit_body, 0)
        plsc.subcore_barrier()

        @pl.when(tid < 10)
        def _():
            pltpu.sync_copy(acc_sh.at[pl.ds(tid * 1000, 1000)],
                            agg_hbm.at[cid].at[pl.ds(tid * 1000, 1000)])

    zeros = jnp.zeros((N0, D), jnp.float32)
    return k(table, src2d, dst2d, zeros)


def _dense_z(h, gs, wc, wf, deg, blk):
    n = h.shape[0]
    ng = len(gs)

    def body(*refs):
        h_ref = refs[0]
        g = refs[1][...]
        if ng == 2:
            g = g + refs[2][...]
        wc_ref, wf_ref, deg_ref, z_ref = refs[1 + ng:]
        acc = jnp.dot(h_ref[...], wc_ref[...], preferred_element_type=jnp.float32)
        acc += jnp.dot(g, wf_ref[...], preferred_element_type=jnp.float32)
        z_ref[...] = acc * lax.rsqrt(deg_ref[...] + 1.0)

    return pl.pallas_call(
        body,
        grid=(n // blk,),
        in_specs=[pl.BlockSpec((blk, D), lambda i: (i, 0))] * (1 + ng) + [
            pl.BlockSpec((D, D), lambda i: (0, 0)),
            pl.BlockSpec((D, D), lambda i: (0, 0)),
            pl.BlockSpec((blk, 1), lambda i: (i, 0)),
        ],
        out_specs=pl.BlockSpec((blk, D), lambda i: (i, 0)),
        out_shape=jax.ShapeDtypeStruct((n, D), jnp.float32),
    )(h, *gs, wc, wf, deg)


def _epilogue(aggs, z, deg, b, gamma, beta, blk):
    n = z.shape[0]
    na = len(aggs)

    def body(*refs):
        agg = refs[0][...]
        if na == 2:
            agg = agg + refs[1][...]
        z_ref, deg_ref, b_ref, gamma_ref, beta_ref, o_ref = refs[na:]
        pre = (agg + z_ref[...]) * lax.rsqrt(deg_ref[...] + 1.0) + b_ref[...]
        mu = jnp.mean(pre, axis=-1, keepdims=True)
        var = jnp.mean((pre - mu) ** 2, axis=-1, keepdims=True)
        ln = (pre - mu) * lax.rsqrt(var + 1e-5) * gamma_ref[...] + beta_ref[...]
        o_ref[...] = jnp.maximum(ln, 0.0)

    return pl.pallas_call(
        body,
        grid=(n // blk,),
        in_specs=[pl.BlockSpec((blk, D), lambda i: (i, 0))] * (na + 1) + [
            pl.BlockSpec((blk, 1), lambda i: (i, 0)),
            pl.BlockSpec((1, D), lambda i: (0, 0)),
            pl.BlockSpec((1, D), lambda i: (0, 0)),
            pl.BlockSpec((1, D), lambda i: (0, 0)),
        ],
        out_specs=pl.BlockSpec((blk, D), lambda i: (i, 0)),
        out_shape=jax.ShapeDtypeStruct((n, D), jnp.float32),
    )(*aggs, z, deg, b, gamma, beta)


def _weights(p):
    wc = p['Wc'] * p['conv_w'][None, :]
    wf = p['Wf'] * p['fuse_w'][None, :]
    b = p['bc'] * p['conv_w'] + p['bf'] * p['fuse_w']
    return wc, wf, b


def _pad_idx(idx, nrow_pad):
    pad = nrow_pad * 128 - idx.shape[0]
    return jnp.concatenate([idx, jnp.zeros((pad,), jnp.int32)]).reshape(
        nrow_pad, 128)


def kernel(h0, h1, edge_index0, edge_index1, params):
    src0, dst0 = edge_index0[0], edge_index0[1]
    src1, dst1 = edge_index1[0], edge_index1[1]

    src0h = jnp.concatenate([src0, jnp.full((3840,), N0, jnp.int32)]).reshape(1280, 128)
    dst0h = jnp.concatenate([dst0, jnp.full((3840,), N0, jnp.int32)]).reshape(1280, 128)
    hist0 = _sc_hist2(src0h, dst0h, 10240)
    hist1 = _sc_hist2(src1.reshape(10000, 128), dst1.reshape(10000, 128), 163840)

    src0p = _pad_idx(src0, 1280)
    dst0p = _pad_idx(dst0, 1280)
    gath, inc2 = _sc_gather_inc(h0, h1, dst0p)

    p = params['td']
    wc, wf, b = _weights(p)
    z0 = _dense_z(h0, (inc2[0], inc2[1]), wc, wf, hist0[0][:N0, None], 400)
    agg0 = _sc_gather_scatter0(z0, src0p, dst0p)
    r0 = _epilogue((agg0[0], agg0[1]), z0, hist0[1][:N0, None], b[None, :],
                   p['gamma'][None, :], p['beta'][None, :], 400)

    p = params['bu']
    wc, wf, b = _weights(p)
    z1 = _dense_z(h1, (gath,), wc, wf, hist1[0][:E0, None], 2000)
    agg1 = jax.ops.segment_sum(jnp.take(z1, src1, axis=0), dst1, num_segments=E0)
    r1 = _epilogue((agg1,), z1, hist1[1][:E0, None], b[None, :],
                   p['gamma'][None, :], p['beta'][None, :], 2000)
    return (r0, r1)


# speedup vs baseline: 3.7940x; 1.3249x over previous
import functools

import jax
import jax.numpy as jnp
from jax import lax
from jax.experimental import pallas as pl
from jax.experimental.pallas import tpu as pltpu
from jax.experimental.pallas import tpu_sc as plsc

N0 = 10000
E0 = 160000
E1 = 1280000
D = 128

_NC = 2
_NS = 16
_NW = _NC * _NS

_mesh = plsc.VectorSubcoreMesh(core_axis_name="c", subcore_axis_name="s")


def _sc_hist2(idx_a, idx_b, nbins_pad):
    nrow = idx_a.shape[0]
    units = nrow // 8
    q, r = units // _NS, units % _NS
    stripe = nbins_pad // _NS

    @functools.partial(
        pl.kernel,
        out_type=jax.ShapeDtypeStruct((2, nbins_pad), jnp.float32),
        mesh=_mesh,
        scratch_types=[
            pltpu.VMEM((8, 128), jnp.int32),
            pltpu.VMEM((128,), jnp.float32),
            pltpu.VMEM((stripe,), jnp.float32),
            pltpu.VMEM_SHARED((nbins_pad,), jnp.float32),
        ],
    )
    def k(a_hbm, b_hbm, out_hbm, idx_v, ones_v, zero_v, acc_sh):
        cid = lax.axis_index("c")
        tid = lax.axis_index("s")

        def fill(i, _):
            zero_v[pl.ds(i * 16, 16)] = jnp.zeros((16,), jnp.float32)
            return 0

        lax.fori_loop(0, stripe // 16, fill, 0)
        for i in range(8):
            ones_v[pl.ds(i * 16, 16)] = jnp.ones((16,), jnp.float32)
        pltpu.sync_copy(zero_v, acc_sh.at[pl.ds(tid * stripe, stripe)])
        plsc.subcore_barrier()

        nunit = q + jnp.where(tid < r, 1, 0)
        base = tid * q + jnp.minimum(tid, r)

        def unit_body(u, _):
            @pl.when(cid == 0)
            def _():
                pltpu.sync_copy(a_hbm.at[pl.ds((base + u) * 8, 8)], idx_v)

            @pl.when(cid == 1)
            def _():
                pltpu.sync_copy(b_hbm.at[pl.ds((base + u) * 8, 8)], idx_v)

            for j in range(8):
                pltpu.sync_copy(ones_v, acc_sh.at[idx_v.at[j]], add=True)
            return 0

        lax.fori_loop(0, nunit, unit_body, 0)
        plsc.subcore_barrier()
        pltpu.sync_copy(acc_sh.at[pl.ds(tid * stripe, stripe)],
                        out_hbm.at[cid].at[pl.ds(tid * stripe, stripe)])

    return k(idx_a, idx_b)


def _sc_gather_inc(h0a, h1a, idx2d):
    nblk_real = E0 // 128
    units_w = idx2d.shape[0] // 8 // _NW

    @functools.partial(
        pl.kernel,
        out_type=(jax.ShapeDtypeStruct((E0, D), jnp.float32),
                  jax.ShapeDtypeStruct((2, N0, D), jnp.float32)),
        mesh=_mesh,
        scratch_types=[
            pltpu.VMEM((8, 128), jnp.int32),
            pltpu.VMEM((128, D), jnp.float32),
            pltpu.VMEM((128, D), jnp.float32),
            pltpu.VMEM_SHARED((N0, D), jnp.float32),
            pltpu.SemaphoreType.DMA,
        ],
    )
    def k(h0_hbm, h1_hbm, idx_hbm, zeros_hbm, gath_hbm, inc_hbm, idx_v,
          grow_v, hrow_v, acc_sh, sem):
        cid = lax.axis_index("c")
        tid = lax.axis_index("s")
        w = tid * _NC + cid

        @pl.when(tid < 10)
        def _():
            pltpu.sync_copy(zeros_hbm.at[pl.ds(tid * 1000, 1000)],
                            acc_sh.at[pl.ds(tid * 1000, 1000)])

        plsc.subcore_barrier()

        def unit_body(u, _):
            ub = w * units_w + u
            pltpu.sync_copy(idx_hbm.at[pl.ds(ub * 8, 8)], idx_v)
            for j in range(8):
                gblk = ub * 8 + j

                @pl.when(gblk < nblk_real)
                def _():
                    pltpu.async_copy(h0_hbm.at[idx_v.at[j]], grow_v, sem).wait()
                    pltpu.sync_copy(grow_v,
                                    gath_hbm.at[pl.ds(gblk * 128, 128)])
                    pltpu.sync_copy(h1_hbm.at[pl.ds(gblk * 128, 128)], hrow_v)
                    pltpu.sync_copy(hrow_v, acc_sh.at[idx_v.at[j]], add=True)
            return 0

        lax.fori_loop(0, units_w, unit_body, 0)
        plsc.subcore_barrier()

        @pl.when(tid < 10)
        def _():
            pltpu.sync_copy(acc_sh.at[pl.ds(tid * 1000, 1000)],
                            inc_hbm.at[cid].at[pl.ds(tid * 1000, 1000)])

    zeros = jnp.zeros((N0, D), jnp.float32)
    return k(h0a, h1a, idx2d, zeros)


def _sc_gather_scatter0(table, src2d, dst2d):
    nblk_real = E0 // 128
    units_w = src2d.shape[0] // 8 // _NW

    @functools.partial(
        pl.kernel,
        out_type=jax.ShapeDtypeStruct((2, N0, D), jnp.float32),
        mesh=_mesh,
        scratch_types=[
            pltpu.VMEM((8, 128), jnp.int32),
            pltpu.VMEM((8, 128), jnp.int32),
            pltpu.VMEM((128, D), jnp.float32),
            pltpu.VMEM_SHARED((N0, D), jnp.float32),
            pltpu.SemaphoreType.DMA,
        ],
    )
    def k(tab_hbm, src_hbm, dst_hbm, zeros_hbm, agg_hbm, src_v, dst_v,
          rows_v, acc_sh, sem):
        cid = lax.axis_index("c")
        tid = lax.axis_index("s")
        w = tid * _NC + cid

        @pl.when(tid < 10)
        def _():
            pltpu.sync_copy(zeros_hbm.at[pl.ds(tid * 1000, 1000)],
                            acc_sh.at[pl.ds(tid * 1000, 1000)])

        plsc.subcore_barrier()

        def unit_body(u, _):
            ub = w * units_w + u
            pltpu.sync_copy(src_hbm.at[pl.ds(ub * 8, 8)], src_v)
            pltpu.sync_copy(dst_hbm.at[pl.ds(ub * 8, 8)], dst_v)
            for j in range(8):
                gblk = ub * 8 + j

                @pl.when(gblk < nblk_real)
                def _():
                    pltpu.async_copy(tab_hbm.at[src_v.at[j]], rows_v, sem).wait()
                    pltpu.sync_copy(rows_v, acc_sh.at[dst_v.at[j]], add=True)
            return 0

        lax.fori_loop(0, units_w, unit_body, 0)
        plsc.subcore_barrier()

        @pl.when(tid < 10)
        def _():
            pltpu.sync_copy(acc_sh.at[pl.ds(tid * 1000, 1000)],
                            agg_hbm.at[cid].at[pl.ds(tid * 1000, 1000)])

    zeros = jnp.zeros((N0, D), jnp.float32)
    return k(table, src2d, dst2d, zeros)


def _sc_gather1(table, idx2d):
    nblk = idx2d.shape[0]
    units = nblk // 8
    q, r = units // _NW, units % _NW

    @functools.partial(
        pl.kernel,
        out_type=jax.ShapeDtypeStruct((nblk * 128, D), jnp.float32),
        mesh=_mesh,
        scratch_types=[
            pltpu.VMEM((8, 128), jnp.int32),
            pltpu.VMEM((2, 128, D), jnp.float32),
            pltpu.SemaphoreType.DMA,
            pltpu.SemaphoreType.DMA,
        ],
    )
    def k(tab_hbm, idx_hbm, out_hbm, idx_v, rows_v, sem0, sem1):
        cid = lax.axis_index("c")
        tid = lax.axis_index("s")
        w = tid * _NC + cid
        nunit = q + jnp.where(w < r, 1, 0)
        ubase = w * q + jnp.minimum(w, r)

        def unit_body(u, _):
            ub = ubase + u
            pltpu.sync_copy(idx_hbm.at[pl.ds(ub * 8, 8)], idx_v)
            for j in range(0, 8, 2):
                gblk = ub * 8 + j
                c0 = pltpu.async_copy(tab_hbm.at[idx_v.at[j]],
                                      rows_v.at[0], sem0)
                c1 = pltpu.async_copy(tab_hbm.at[idx_v.at[j + 1]],
                                      rows_v.at[1], sem1)
                c0.wait()
                pltpu.sync_copy(rows_v.at[0],
                                out_hbm.at[pl.ds(gblk * 128, 128)])
                c1.wait()
                pltpu.sync_copy(rows_v.at[1],
                                out_hbm.at[pl.ds((gblk + 1) * 128, 128)])
            return 0

        lax.fori_loop(0, nunit, unit_body, 0)

    return k(table, idx2d)


def _dense_z(h, gs, wc, wf, deg, blk):
    n = h.shape[0]
    ng = len(gs)

    def body(*refs):
        h_ref = refs[0]
        g = refs[1][...]
        if ng == 2:
            g = g + refs[2][...]
        wc_ref, wf_ref, deg_ref, z_ref = refs[1 + ng:]
        acc = jnp.dot(h_ref[...], wc_ref[...], preferred_element_type=jnp.float32)
        acc += jnp.dot(g, wf_ref[...], preferred_element_type=jnp.float32)
        z_ref[...] = acc * lax.rsqrt(deg_ref[...] + 1.0)

    return pl.pallas_call(
        body,
        grid=(n // blk,),
        in_specs=[pl.BlockSpec((blk, D), lambda i: (i, 0))] * (1 + ng) + [
            pl.BlockSpec((D, D), lambda i: (0, 0)),
            pl.BlockSpec((D, D), lambda i: (0, 0)),
            pl.BlockSpec((blk, 1), lambda i: (i, 0)),
        ],
        out_specs=pl.BlockSpec((blk, D), lambda i: (i, 0)),
        out_shape=jax.ShapeDtypeStruct((n, D), jnp.float32),
    )(h, *gs, wc, wf, deg)


def _epilogue(aggs, z, deg, b, gamma, beta, blk):
    n = z.shape[0]
    na = len(aggs)

    def body(*refs):
        agg = refs[0][...]
        if na == 2:
            agg = agg + refs[1][...]
        z_ref, deg_ref, b_ref, gamma_ref, beta_ref, o_ref = refs[na:]
        pre = (agg + z_ref[...]) * lax.rsqrt(deg_ref[...] + 1.0) + b_ref[...]
        mu = jnp.mean(pre, axis=-1, keepdims=True)
        var = jnp.mean((pre - mu) ** 2, axis=-1, keepdims=True)
        ln = (pre - mu) * lax.rsqrt(var + 1e-5) * gamma_ref[...] + beta_ref[...]
        o_ref[...] = jnp.maximum(ln, 0.0)

    return pl.pallas_call(
        body,
        grid=(n // blk,),
        in_specs=[pl.BlockSpec((blk, D), lambda i: (i, 0))] * (na + 1) + [
            pl.BlockSpec((blk, 1), lambda i: (i, 0)),
            pl.BlockSpec((1, D), lambda i: (0, 0)),
            pl.BlockSpec((1, D), lambda i: (0, 0)),
            pl.BlockSpec((1, D), lambda i: (0, 0)),
        ],
        out_specs=pl.BlockSpec((blk, D), lambda i: (i, 0)),
        out_shape=jax.ShapeDtypeStruct((n, D), jnp.float32),
    )(*aggs, z, deg, b, gamma, beta)


def _weights(p):
    wc = p['Wc'] * p['conv_w'][None, :]
    wf = p['Wf'] * p['fuse_w'][None, :]
    b = p['bc'] * p['conv_w'] + p['bf'] * p['fuse_w']
    return wc, wf, b


def _pad_idx(idx, nrow_pad):
    pad = nrow_pad * 128 - idx.shape[0]
    return jnp.concatenate([idx, jnp.zeros((pad,), jnp.int32)]).reshape(
        nrow_pad, 128)


def kernel(h0, h1, edge_index0, edge_index1, params):
    src0, dst0 = edge_index0[0], edge_index0[1]
    src1, dst1 = edge_index1[0], edge_index1[1]

    src0h = jnp.concatenate([src0, jnp.full((3840,), N0, jnp.int32)]).reshape(1280, 128)
    dst0h = jnp.concatenate([dst0, jnp.full((3840,), N0, jnp.int32)]).reshape(1280, 128)
    hist0 = _sc_hist2(src0h, dst0h, 10240)
    hist1 = _sc_hist2(src1.reshape(10000, 128), dst1.reshape(10000, 128), 163840)

    src0p = _pad_idx(src0, 1280)
    dst0p = _pad_idx(dst0, 1280)
    gath, inc2 = _sc_gather_inc(h0, h1, dst0p)

    p = params['td']
    wc, wf, b = _weights(p)
    z0 = _dense_z(h0, (inc2[0], inc2[1]), wc, wf, hist0[0][:N0, None], 400)
    agg0 = _sc_gather_scatter0(z0, src0p, dst0p)
    r0 = _epilogue((agg0[0], agg0[1]), z0, hist0[1][:N0, None], b[None, :],
                   p['gamma'][None, :], p['beta'][None, :], 400)

    p = params['bu']
    wc, wf, b = _weights(p)
    z1 = _dense_z(h1, (gath,), wc, wf, hist1[0][:E0, None], 2000)
    z1s = _sc_gather1(z1, src1.reshape(10000, 128))
    agg1 = jax.ops.segment_sum(z1s, dst1, num_segments=E0)
    r1 = _epilogue((agg1,), z1, hist1[1][:E0, None], b[None, :],
                   p['gamma'][None, :], p['beta'][None, :], 2000)
    return (r0, r1)
